# baseline (speedup 1.0000x reference)
import functools

import jax
import jax.numpy as jnp
from jax import lax
from jax.experimental import pallas as pl
from jax.experimental.pallas import tpu as pltpu
from jax.experimental.pallas import tpu_sc as plsc

B = 16384
U = 1000000
I = 1000000
H = 16
RPW = 128 // H
NC = 2
NS = 16
L = 16
NW = NC * NS
BPW = B // NW
CH = 128
NCH = BPW // CH
NBC = CH // L

_mesh = plsc.VectorSubcoreMesh(core_axis_name="c", subcore_axis_name="s",
                               num_cores=NC, num_subcores=NS)


@functools.partial(
    pl.kernel,
    out_type=(
        jax.ShapeDtypeStruct((B,), jnp.float32),
        jax.ShapeDtypeStruct((NW, L), jnp.float32),
    ),
    mesh=_mesh,
    compiler_params=pltpu.CompilerParams(needs_layout_passes=False,
                                         use_tc_tiling_on_sc=True),
    scratch_types=[
        pltpu.VMEM((NCH, CH), jnp.int32),
        pltpu.VMEM((NCH, CH), jnp.int32),
        pltpu.VMEM((NCH, CH), jnp.int32),
        pltpu.VMEM((NCH, CH), jnp.int32),
        pltpu.VMEM((BPW,), jnp.int32),
        pltpu.VMEM((BPW,), jnp.int32),
        pltpu.VMEM((CH, 128), jnp.float32),
        pltpu.VMEM((CH, 128), jnp.float32),
        pltpu.VMEM((CH, 128), jnp.float32),
        pltpu.VMEM((CH, 128), jnp.float32),
        pltpu.VMEM((BPW,), jnp.float32),
        pltpu.VMEM((BPW,), jnp.float32),
        pltpu.VMEM((BPW,), jnp.float32),
        pltpu.VMEM((BPW,), jnp.float32),
        pltpu.VMEM((L,), jnp.float32),
        pltpu.VMEM((L,), jnp.float32),
        pltpu.SemaphoreType.DMA,
        pltpu.SemaphoreType.DMA,
        pltpu.SemaphoreType.DMA,
    ],
)
def _mf_sc_kernel(uhi_h, ihi_h, uor_h, ior_h, uoff_h, ioff_h, rating_h,
                  uw_h, iw_h, ub_h, ib_h, bias_h,
                  tgt_h, part_h,
                  uhi_v, ihi_v, uor_v, ior_v, uoff_v, ioff_v,
                  ubuf0, ubuf1, ibuf0, ibuf1,
                  ubr_v, ibr_v, rat_v, out_v, sqa_v, bias_v,
                  semA, semB0, semB1):
    wid = lax.axis_index("s") * NC + lax.axis_index("c")
    base = wid * BPW

    pltpu.sync_copy(uhi_h.at[pl.ds(wid * NCH, NCH)], uhi_v)
    pltpu.sync_copy(ihi_h.at[pl.ds(wid * NCH, NCH)], ihi_v)
    pltpu.sync_copy(uor_h.at[pl.ds(wid * NCH, NCH)], uor_v)
    pltpu.sync_copy(ior_h.at[pl.ds(wid * NCH, NCH)], ior_v)
    pltpu.sync_copy(uoff_h.at[pl.ds(base, BPW)], uoff_v)
    pltpu.sync_copy(ioff_h.at[pl.ds(base, BPW)], ioff_v)
    pltpu.sync_copy(rating_h.at[pl.ds(base, BPW)], rat_v)
    pltpu.sync_copy(bias_h, bias_v)

    bias_copies = []
    for c in range(NCH):
        sl = pl.ds(c * CH, CH)
        bias_copies.append(pltpu.async_copy(ub_h.at[uor_v.at[c]], ubr_v.at[sl], semA))
        bias_copies.append(pltpu.async_copy(ib_h.at[ior_v.at[c]], ibr_v.at[sl], semA))

    ubufs = (ubuf0, ubuf1)
    ibufs = (ibuf0, ibuf1)
    sems = (semB0, semB1)

    def fire(c):
        p = c % 2
        return (pltpu.async_copy(uw_h.at[uhi_v.at[c]], ubufs[p], sems[p]),
                pltpu.async_copy(iw_h.at[ihi_v.at[c]], ibufs[p], sems[p]))

    pending = fire(0)
    for cp in bias_copies:
        cp.wait()

    gbias = bias_v[...]
    lanes = lax.iota(jnp.int32, L)
    sqacc = jnp.zeros((L,), jnp.float32)

    for c in range(NCH):
        p = c % 2
        for cp in pending:
            cp.wait()
        if c + 1 < NCH:
            pending = fire(c + 1)
        ubuf, ibuf = ubufs[p], ibufs[p]
        for j in range(NBC):
            o = c * CH + j * L
            rows = j * L + lanes
            ucol0 = uoff_v[pl.ds(o, L)]
            icol0 = ioff_v[pl.ds(o, L)]
            ub = ubr_v[pl.ds(o, L)]
            ib = ibr_v[pl.ds(o, L)]
            acc = jnp.zeros((L,), jnp.float32)
            for h in range(H):
                gu = plsc.load_gather(ubuf, [rows, ucol0 + h])
                gi = plsc.load_gather(ibuf, [rows, icol0 + h])
                acc = acc + (gu + ub) * (gi + ib)
            out = acc + gbias
            out_v[pl.ds(o, L)] = out
            err = out - rat_v[pl.ds(o, L)]
            sqacc = sqacc + err * err

    sqa_v[...] = sqacc
    pltpu.sync_copy(sqa_v, part_h.at[wid])
    pltpu.sync_copy(out_v, tgt_h.at[pl.ds(base, BPW)])


def kernel(user, item, rating, user_weight, item_weight, user_bias, item_bias,
           bias):
    user = user.astype(jnp.int32)
    item = item.astype(jnp.int32)
    uhi = (user >> 3).reshape(NW * NCH, CH)
    ihi = (item >> 3).reshape(NW * NCH, CH)
    uoff = (user & 7) << 4
    ioff = (item & 7) << 4
    uor = user.reshape(NW * NCH, CH)
    ior = item.reshape(NW * NCH, CH)
    uw128 = user_weight.reshape(U // RPW, 128)
    iw128 = item_weight.reshape(I // RPW, 128)
    bias16 = jnp.broadcast_to(bias.astype(jnp.float32), (L,))
    target, parts = _mf_sc_kernel(uhi, ihi, uor, ior, uoff, ioff, rating,
                                  uw128, iw128, user_bias.reshape(U),
                                  item_bias.reshape(I), bias16)
    loss = jnp.sum(parts) / B
    return (target, loss)
